# baseline (speedup 1.0000x reference)
import functools

import jax
import jax.numpy as jnp
from jax import lax
from jax.experimental import pallas as pl
from jax.experimental.pallas import tpu as pltpu
from jax.experimental.pallas import tpu_sc as plsc

VOCAB = 1_000_000
EMBED_DIM = 32
BATCH = 16384
HIST_LEN = 50
TOTAL = BATCH * HIST_LEN

NUM_CORES = 2
NUM_SUBCORES = 16
NW = NUM_CORES * NUM_SUBCORES
LB = 128
NBLK = BATCH // LB
BLK_PER_W = NBLK // NW
CHUNK = LB
NCHUNK = BLK_PER_W * HIST_LEN
NBUF = 2
FBLK = EMBED_DIM // 8

_mesh = plsc.VectorSubcoreMesh(core_axis_name="c", subcore_axis_name="s")


@functools.partial(
    pl.kernel,
    mesh=_mesh,
    out_type=jax.ShapeDtypeStruct(
        (HIST_LEN, FBLK, NBLK, 8, LB), jnp.float32
    ),
    scratch_types=(
        [pltpu.VMEM((NCHUNK, CHUNK), jnp.int32)]
        + [pltpu.VMEM((CHUNK, EMBED_DIM), jnp.float32) for _ in range(NBUF)]
        + [pltpu.VMEM((FBLK, 8, LB), jnp.float32) for _ in range(NBUF)]
        + [pltpu.SemaphoreType.DMA((NBUF,)), pltpu.SemaphoreType.DMA((NBUF,))]
    ),
    compiler_params=pltpu.CompilerParams(
        use_tc_tiling_on_sc=False, needs_layout_passes=False
    ),
)
def _sc_gather(table_hbm, idx_hbm, out_hbm, idx_v, *rest):
    rows_bufs = rest[:NBUF]
    tile_bufs = rest[NBUF:2 * NBUF]
    gsems, wsems = rest[2 * NBUF], rest[2 * NBUF + 1]

    wid = lax.axis_index("s") * NUM_CORES + lax.axis_index("c")

    pltpu.sync_copy(idx_hbm.at[wid], idx_v)

    lane = lax.iota(jnp.int32, 16)
    row_vecs = [lane + (16 * k) for k in range(CHUNK // 16)]
    col_vecs = [jnp.full((16,), f, jnp.int32) for f in range(EMBED_DIM)]

    def fire(j, b):
        pltpu.async_copy(table_hbm.at[idx_v.at[j]], rows_bufs[b], gsems.at[b])

    for b in range(NBUF):
        fire(b, b)

    def round_body(g, _):
        for b in range(NBUF):
            j = g * NBUF + b
            c = wid * BLK_PER_W + j // HIST_LEN
            h = j % HIST_LEN

            pltpu.make_async_copy(
                table_hbm.at[idx_v.at[j]], rows_bufs[b], gsems.at[b]
            ).wait()

            @pl.when(j >= NBUF)
            def _():
                for a in range(FBLK):
                    pltpu.make_async_copy(
                        tile_bufs[b].at[a], out_hbm.at[0, 0, 0], wsems.at[b]
                    ).wait()

            src = rows_bufs[b]
            dst = tile_bufs[b]
            for k in range(CHUNK // 16):
                for f in range(EMBED_DIM):
                    vals = plsc.load_gather(src, [row_vecs[k], col_vecs[f]])
                    dst[f // 8, f % 8, pl.ds(16 * k, 16)] = vals

            for a in range(FBLK):
                pltpu.async_copy(
                    tile_bufs[b].at[a], out_hbm.at[h, a, c], wsems.at[b]
                )

            @pl.when(j + NBUF < NCHUNK)
            def _():
                fire(j + NBUF, b)

        return _

    lax.fori_loop(0, NCHUNK // NBUF, round_body, None)

    for b in range(NBUF):
        for a in range(FBLK):
            pltpu.make_async_copy(
                tile_bufs[b].at[a], out_hbm.at[0, 0, 0], wsems.at[b]
            ).wait()


def kernel(params, inputs):
    idx = inputs.reshape(NBLK, LB, HIST_LEN).transpose(0, 2, 1)
    idx = idx.reshape(NW, NCHUNK, CHUNK)
    out5 = _sc_gather(params, idx)
    return out5.transpose(2, 4, 0, 1, 3).reshape(BATCH, HIST_LEN, EMBED_DIM)


# speedup vs baseline: 1.2858x; 1.2858x over previous
import functools

import jax
import jax.numpy as jnp
from jax import lax
from jax.experimental import pallas as pl
from jax.experimental.pallas import tpu as pltpu
from jax.experimental.pallas import tpu_sc as plsc

VOCAB = 1_000_000
EMBED_DIM = 32
BATCH = 16384
HIST_LEN = 50
TOTAL = BATCH * HIST_LEN

NUM_CORES = 2
NUM_SUBCORES = 16
NW = NUM_CORES * NUM_SUBCORES
LB = 128
NBLK = BATCH // LB
BLK_PER_W = NBLK // NW
CHUNK = LB
NCHUNK = BLK_PER_W * HIST_LEN
NBUF = 2
FBLK = EMBED_DIM // 8

_mesh = plsc.VectorSubcoreMesh(core_axis_name="c", subcore_axis_name="s")


@functools.partial(
    pl.kernel,
    mesh=_mesh,
    out_type=jax.ShapeDtypeStruct(
        (HIST_LEN, FBLK, NBLK, 8, LB), jnp.float32
    ),
    scratch_types=(
        [pltpu.VMEM((NCHUNK, CHUNK), jnp.int32)]
        + [pltpu.VMEM((CHUNK, EMBED_DIM), jnp.float32) for _ in range(NBUF)]
        + [pltpu.VMEM((FBLK, 8, LB), jnp.float32) for _ in range(NBUF)]
        + [pltpu.SemaphoreType.DMA((NBUF,)), pltpu.SemaphoreType.DMA((NBUF,))]
    ),
    compiler_params=pltpu.CompilerParams(
        use_tc_tiling_on_sc=False, needs_layout_passes=False
    ),
)
def _sc_gather(table_hbm, idx_hbm, out_hbm, idx_v, *rest):
    rows_bufs = rest[:NBUF]
    tile_bufs = rest[NBUF:2 * NBUF]
    gsems, wsems = rest[2 * NBUF], rest[2 * NBUF + 1]

    wid = lax.axis_index("s") * NUM_CORES + lax.axis_index("c")

    pltpu.sync_copy(idx_hbm.at[wid], idx_v)

    lane = lax.iota(jnp.int32, 16)
    row_vecs = [lane + (16 * k) for k in range(CHUNK // 16)]
    col_vecs = [jnp.full((16,), f, jnp.int32) for f in range(EMBED_DIM)]

    def fire(j, b):
        pltpu.async_copy(table_hbm.at[idx_v.at[j]], rows_bufs[b], gsems.at[b])

    for b in range(NBUF):
        fire(b, b)

    def round_body(g, _):
        for b in range(NBUF):
            j = g * NBUF + b
            c = wid * BLK_PER_W + j // HIST_LEN
            h = j % HIST_LEN

            pltpu.make_async_copy(
                table_hbm.at[idx_v.at[j]], rows_bufs[b], gsems.at[b]
            ).wait()

            @pl.when(j >= NBUF)
            def _():
                for a in range(FBLK):
                    pltpu.make_async_copy(
                        tile_bufs[b].at[a], out_hbm.at[0, 0, 0], wsems.at[b]
                    ).wait()

            src = rows_bufs[b]
            dst = tile_bufs[b]

            @plsc.parallel_loop(0, CHUNK // 16, 1, unroll=4)
            def _(k):
                rows = lane + 16 * k
                for f in range(EMBED_DIM):
                    vals = plsc.load_gather(src, [rows, col_vecs[f]])
                    dst[f // 8, f % 8, pl.ds(16 * k, 16)] = vals

            for a in range(FBLK):
                pltpu.async_copy(
                    tile_bufs[b].at[a], out_hbm.at[h, a, c], wsems.at[b]
                )

            @pl.when(j + NBUF < NCHUNK)
            def _():
                fire(j + NBUF, b)

        return _

    lax.fori_loop(0, NCHUNK // NBUF, round_body, None)

    for b in range(NBUF):
        for a in range(FBLK):
            pltpu.make_async_copy(
                tile_bufs[b].at[a], out_hbm.at[0, 0, 0], wsems.at[b]
            ).wait()


def kernel(params, inputs):
    idx = inputs.reshape(NBLK, LB, HIST_LEN).transpose(0, 2, 1)
    idx = idx.reshape(NW, NCHUNK, CHUNK)
    out5 = _sc_gather(params, idx)
    return out5.transpose(2, 4, 0, 1, 3).reshape(BATCH, HIST_LEN, EMBED_DIM)


# speedup vs baseline: 1.2862x; 1.0004x over previous
import functools

import jax
import jax.numpy as jnp
from jax import lax
from jax.experimental import pallas as pl
from jax.experimental.pallas import tpu as pltpu
from jax.experimental.pallas import tpu_sc as plsc

VOCAB = 1_000_000
EMBED_DIM = 32
BATCH = 16384
HIST_LEN = 50
TOTAL = BATCH * HIST_LEN

NUM_CORES = 2
NUM_SUBCORES = 16
NW = NUM_CORES * NUM_SUBCORES
LB = 128
NBLK = BATCH // LB
BLK_PER_W = NBLK // NW
CHUNK = LB
NCHUNK = BLK_PER_W * HIST_LEN
NBUF = 4
FBLK = EMBED_DIM // 8

_mesh = plsc.VectorSubcoreMesh(core_axis_name="c", subcore_axis_name="s")


@functools.partial(
    pl.kernel,
    mesh=_mesh,
    out_type=jax.ShapeDtypeStruct(
        (HIST_LEN, FBLK, NBLK, 8, LB), jnp.float32
    ),
    scratch_types=(
        [pltpu.VMEM((NCHUNK, CHUNK), jnp.int32)]
        + [pltpu.VMEM((CHUNK, EMBED_DIM), jnp.float32) for _ in range(NBUF)]
        + [pltpu.VMEM((FBLK, 8, LB), jnp.float32) for _ in range(NBUF)]
        + [pltpu.SemaphoreType.DMA((NBUF,)), pltpu.SemaphoreType.DMA((NBUF,))]
    ),
    compiler_params=pltpu.CompilerParams(
        use_tc_tiling_on_sc=False, needs_layout_passes=False
    ),
)
def _sc_gather(table_hbm, idx_hbm, out_hbm, idx_v, *rest):
    rows_bufs = rest[:NBUF]
    tile_bufs = rest[NBUF:2 * NBUF]
    gsems, wsems = rest[2 * NBUF], rest[2 * NBUF + 1]

    wid = lax.axis_index("s") * NUM_CORES + lax.axis_index("c")

    pltpu.sync_copy(idx_hbm.at[wid], idx_v)

    lane = lax.iota(jnp.int32, 16)
    row_vecs = [lane + (16 * k) for k in range(CHUNK // 16)]
    col_vecs = [jnp.full((16,), f, jnp.int32) for f in range(EMBED_DIM)]

    def fire(j, b):
        pltpu.async_copy(table_hbm.at[idx_v.at[j]], rows_bufs[b], gsems.at[b])

    for b in range(NBUF):
        fire(b, b)

    def round_body(g, _):
        for b in range(NBUF):
            j = g * NBUF + b
            c = wid * BLK_PER_W + j // HIST_LEN
            h = j % HIST_LEN

            pltpu.make_async_copy(
                table_hbm.at[idx_v.at[j]], rows_bufs[b], gsems.at[b]
            ).wait()

            @pl.when(j >= NBUF)
            def _():
                for a in range(FBLK):
                    pltpu.make_async_copy(
                        tile_bufs[b].at[a], out_hbm.at[0, 0, 0], wsems.at[b]
                    ).wait()

            src = rows_bufs[b]
            dst = tile_bufs[b]

            @plsc.parallel_loop(0, CHUNK // 16, 1, unroll=4)
            def _(k):
                rows = lane + 16 * k
                for f in range(EMBED_DIM):
                    vals = plsc.load_gather(src, [rows, col_vecs[f]])
                    dst[f // 8, f % 8, pl.ds(16 * k, 16)] = vals

            for a in range(FBLK):
                pltpu.async_copy(
                    tile_bufs[b].at[a], out_hbm.at[h, a, c], wsems.at[b]
                )

            @pl.when(j + NBUF < NCHUNK)
            def _():
                fire(j + NBUF, b)

        return _

    lax.fori_loop(0, NCHUNK // NBUF, round_body, None)

    for b in range(NBUF):
        for a in range(FBLK):
            pltpu.make_async_copy(
                tile_bufs[b].at[a], out_hbm.at[0, 0, 0], wsems.at[b]
            ).wait()


def kernel(params, inputs):
    idx = inputs.reshape(NBLK, LB, HIST_LEN).transpose(0, 2, 1)
    idx = idx.reshape(NW, NCHUNK, CHUNK)
    out5 = _sc_gather(params, idx)
    return out5.transpose(2, 4, 0, 1, 3).reshape(BATCH, HIST_LEN, EMBED_DIM)


# speedup vs baseline: 1.4753x; 1.1470x over previous
import functools

import jax
import jax.numpy as jnp
from jax import lax
from jax.experimental import pallas as pl
from jax.experimental.pallas import tpu as pltpu
from jax.experimental.pallas import tpu_sc as plsc

VOCAB = 1_000_000
EMBED_DIM = 32
BATCH = 16384
HIST_LEN = 50
TOTAL = BATCH * HIST_LEN

NUM_CORES = 2
NUM_SUBCORES = 16
NW = NUM_CORES * NUM_SUBCORES
LB = 128
NBLK = BATCH // LB
BLK_PER_W = NBLK // NW
CHUNK = LB
NCHUNK = BLK_PER_W * HIST_LEN
NBUF = 4
FBLK = EMBED_DIM // 8

_mesh = plsc.VectorSubcoreMesh(core_axis_name="c", subcore_axis_name="s")


@functools.partial(
    pl.kernel,
    mesh=_mesh,
    out_type=jax.ShapeDtypeStruct(
        (HIST_LEN, FBLK, NBLK, 8 * LB), jnp.float32
    ),
    scratch_types=(
        [pltpu.VMEM((NCHUNK, CHUNK), jnp.int32)]
        + [pltpu.VMEM((CHUNK, EMBED_DIM), jnp.float32) for _ in range(NBUF)]
        + [pltpu.VMEM((FBLK * 8 * LB,), jnp.float32) for _ in range(NBUF)]
        + [pltpu.SemaphoreType.DMA((NBUF,)), pltpu.SemaphoreType.DMA((NBUF,))]
    ),
    compiler_params=pltpu.CompilerParams(
        use_tc_tiling_on_sc=False, needs_layout_passes=False
    ),
)
def _sc_gather(table_hbm, idx_hbm, out_hbm, idx_v, *rest):
    rows_bufs = rest[:NBUF]
    tile_bufs = rest[NBUF:2 * NBUF]
    gsems, wsems = rest[2 * NBUF], rest[2 * NBUF + 1]

    wid = lax.axis_index("s") * NUM_CORES + lax.axis_index("c")

    pltpu.sync_copy(idx_hbm.at[wid], idx_v)

    lane = lax.iota(jnp.int32, 16)
    perms = [jnp.bitwise_and(lane + i, 15) for i in range(16)]

    def fire(j, b):
        pltpu.async_copy(table_hbm.at[idx_v.at[j]], rows_bufs[b], gsems.at[b])

    for b in range(NBUF):
        fire(b, b)

    def round_body(g, _):
        for b in range(NBUF):
            j = g * NBUF + b
            c = wid * BLK_PER_W + j // HIST_LEN
            h = j % HIST_LEN

            pltpu.make_async_copy(
                table_hbm.at[idx_v.at[j]], rows_bufs[b], gsems.at[b]
            ).wait()

            @pl.when(j >= NBUF)
            def _():
                for a in range(FBLK):
                    pltpu.make_async_copy(
                        tile_bufs[b].at[pl.ds(a * 8 * LB, 8 * LB)],
                        out_hbm.at[0, 0, 0],
                        wsems.at[b],
                    ).wait()

            src = rows_bufs[b]
            dst = tile_bufs[b]

            @plsc.parallel_loop(0, CHUNK // 16, 1, unroll=2)
            def _(k):
                rows = lane + 16 * k
                for f0 in range(0, EMBED_DIM, 16):
                    for i in range(16):
                        colv = perms[i] + f0 if f0 else perms[i]
                        vals = plsc.load_gather(src, [rows, colv])
                        flat = lax.shift_left(colv, 7) + rows
                        plsc.store_scatter(dst, [flat], vals)

            for a in range(FBLK):
                pltpu.async_copy(
                    tile_bufs[b].at[pl.ds(a * 8 * LB, 8 * LB)],
                    out_hbm.at[h, a, c],
                    wsems.at[b],
                )

            @pl.when(j + NBUF < NCHUNK)
            def _():
                fire(j + NBUF, b)

        return _

    lax.fori_loop(0, NCHUNK // NBUF, round_body, None)

    for b in range(NBUF):
        for a in range(FBLK):
            pltpu.make_async_copy(
                tile_bufs[b].at[pl.ds(a * 8 * LB, 8 * LB)],
                out_hbm.at[0, 0, 0],
                wsems.at[b],
            ).wait()


def kernel(params, inputs):
    idx = inputs.reshape(NBLK, LB, HIST_LEN).transpose(0, 2, 1)
    idx = idx.reshape(NW, NCHUNK, CHUNK)
    out4 = _sc_gather(params, idx)
    out5 = out4.reshape(HIST_LEN, FBLK, NBLK, 8, LB)
    return out5.transpose(2, 4, 0, 1, 3).reshape(BATCH, HIST_LEN, EMBED_DIM)


# speedup vs baseline: 2.0867x; 1.4144x over previous
import functools

import jax
import jax.numpy as jnp
from jax import lax
from jax.experimental import pallas as pl
from jax.experimental.pallas import tpu as pltpu
from jax.experimental.pallas import tpu_sc as plsc

VOCAB = 1_000_000
EMBED_DIM = 32
BATCH = 16384
HIST_LEN = 50
TOTAL = BATCH * HIST_LEN

NUM_CORES = 2
NUM_SUBCORES = 16
NW = NUM_CORES * NUM_SUBCORES
LB = 128
NBLK = BATCH // LB
BLK_PER_W = NBLK // NW
CHUNK = LB
NCHUNK = BLK_PER_W * HIST_LEN
NBUF = 4
FBLK = EMBED_DIM // 8

_mesh = plsc.VectorSubcoreMesh(core_axis_name="c", subcore_axis_name="s")


@functools.partial(
    pl.kernel,
    mesh=_mesh,
    out_type=jax.ShapeDtypeStruct(
        (HIST_LEN, FBLK, NBLK, 8 * LB), jnp.float32
    ),
    scratch_types=(
        [pltpu.VMEM((NCHUNK, CHUNK), jnp.int32)]
        + [pltpu.VMEM((CHUNK, EMBED_DIM), jnp.float32) for _ in range(NBUF)]
        + [pltpu.VMEM((FBLK * 8 * LB,), jnp.float32) for _ in range(NBUF)]
        + [pltpu.SemaphoreType.DMA((NBUF,)), pltpu.SemaphoreType.DMA((NBUF,))]
    ),
    compiler_params=pltpu.CompilerParams(
        use_tc_tiling_on_sc=False, needs_layout_passes=False
    ),
)
def _sc_gather(table_hbm, idx_hbm, out_hbm, idx_v, *rest):
    rows_bufs = rest[:NBUF]
    tile_bufs = rest[NBUF:2 * NBUF]
    gsems, wsems = rest[2 * NBUF], rest[2 * NBUF + 1]

    wid = lax.axis_index("s") * NUM_CORES + lax.axis_index("c")

    pltpu.sync_copy(idx_hbm.at[wid], idx_v)

    lane = lax.iota(jnp.int32, 16)
    perms = [jnp.bitwise_and(lane + i, 15) for i in range(16)]
    pbases = [lax.shift_left(p, 7) for p in perms]

    def fire(j, b):
        pltpu.async_copy(table_hbm.at[idx_v.at[j]], rows_bufs[b], gsems.at[b])

    for b in range(NBUF):
        fire(b, b)

    def round_body(g, _):
        for b in range(NBUF):
            j = g * NBUF + b
            c = wid * BLK_PER_W + j // HIST_LEN
            h = j % HIST_LEN

            pltpu.make_async_copy(
                table_hbm.at[idx_v.at[j]], rows_bufs[b], gsems.at[b]
            ).wait()

            @pl.when(j >= NBUF)
            def _():
                for a in range(FBLK):
                    pltpu.make_async_copy(
                        tile_bufs[b].at[pl.ds(a * 8 * LB, 8 * LB)],
                        out_hbm.at[0, 0, 0],
                        wsems.at[b],
                    ).wait()

            src = rows_bufs[b]
            dst = tile_bufs[b]

            @plsc.parallel_loop(0, CHUNK // 16, 1, unroll=4)
            def _(k):
                rows = lane + 16 * k
                rows2 = rows + (16 << 7)
                for f0 in range(0, EMBED_DIM, 16):
                    for i in range(16):
                        colv = perms[i] + f0 if f0 else perms[i]
                        vals = plsc.load_gather(src, [rows, colv])
                        flat = pbases[i] + (rows2 if f0 else rows)
                        plsc.store_scatter(dst, [flat], vals)

            for a in range(FBLK):
                pltpu.async_copy(
                    tile_bufs[b].at[pl.ds(a * 8 * LB, 8 * LB)],
                    out_hbm.at[h, a, c],
                    wsems.at[b],
                )

            @pl.when(j + NBUF < NCHUNK)
            def _():
                fire(j + NBUF, b)

        return _

    lax.fori_loop(0, NCHUNK // NBUF, round_body, None)

    for b in range(NBUF):
        for a in range(FBLK):
            pltpu.make_async_copy(
                tile_bufs[b].at[pl.ds(a * 8 * LB, 8 * LB)],
                out_hbm.at[0, 0, 0],
                wsems.at[b],
            ).wait()


def kernel(params, inputs):
    idx = inputs.reshape(NBLK, LB, HIST_LEN).transpose(0, 2, 1)
    idx = idx.reshape(NW, NCHUNK, CHUNK)
    out4 = _sc_gather(params, idx)
    out5 = out4.reshape(HIST_LEN, FBLK, NBLK, 8, LB)
    return out5.transpose(2, 4, 0, 1, 3).reshape(BATCH, HIST_LEN, EMBED_DIM)
